# baseline (speedup 1.0000x reference)
import jax
import jax.numpy as jnp
from jax.experimental import pallas as pl
from jax.experimental.pallas import tpu as pltpu

N_CLUSTERS = 1024
N_FEATURES = 256
BATCH = 16384

TILE = 2048
NT = BATCH // TILE
W = 128
NCHUNK = N_CLUSTERS // W


def _fused_body(x_ref, c_ref, assign_ref, counts_ref, sums_ref):
    i = pl.program_id(0)
    x = x_ref[...]

    x2 = jnp.sum(x * x, axis=1, keepdims=True)

    best = None
    bcol = None
    for j in range(NCHUNK):
        cj = c_ref[pl.ds(j * W, W), :]
        c2j = jnp.sum(cj * cj, axis=1)[None, :]
        crossj = jax.lax.dot_general(
            x, cj, (((1,), (1,)), ((), ())),
            preferred_element_type=jnp.float32)
        dj = x2 + c2j - 2.0 * crossj
        if j == 0:
            best = dj
            bcol = jnp.zeros(dj.shape, jnp.float32)
        else:
            lt = dj < best
            best = jnp.where(lt, dj, best)
            bcol = jnp.where(lt, jnp.float32(j), bcol)

    dmin = jnp.min(best, axis=1, keepdims=True)
    lanef = jax.lax.broadcasted_iota(jnp.int32, best.shape, 1).astype(
        jnp.float32)
    big = jnp.float32(N_CLUSTERS)
    cand = jnp.where(best == dmin, bcol * jnp.float32(W) + lanef, big)
    assignf = jnp.min(cand, axis=1, keepdims=True)
    assign_ref[...] = assignf.astype(jnp.int32)

    oh_lane = cand == assignf

    @pl.when(i == 0)
    def _init():
        counts_ref[...] = jnp.zeros_like(counts_ref)
        sums_ref[...] = jnp.zeros_like(sums_ref)

    x_bf = x.astype(jnp.bfloat16)
    for j in range(NCHUNK):
        cond = oh_lane & (bcol == jnp.float32(j))
        ohj = jnp.where(cond, 1.0, 0.0)
        ohj_b = ohj.astype(jnp.bfloat16)
        counts_ref[:, pl.ds(j * W, W)] += jnp.sum(ohj, axis=0)[None, :]
        sums_ref[pl.ds(j * W, W), :] += jax.lax.dot_general(
            ohj_b, x_bf, (((0,), (0,)), ((), ())),
            preferred_element_type=jnp.float32)


@jax.jit
def kernel(batch, centroids):
    assign3, counts2, sums = pl.pallas_call(
        _fused_body,
        grid=(NT,),
        in_specs=[
            pl.BlockSpec((TILE, N_FEATURES), lambda i: (i, 0)),
            pl.BlockSpec((N_CLUSTERS, N_FEATURES), lambda i: (0, 0)),
        ],
        out_specs=[
            pl.BlockSpec((TILE, 1), lambda i: (i, 0)),
            pl.BlockSpec((1, N_CLUSTERS), lambda i: (0, 0)),
            pl.BlockSpec((N_CLUSTERS, N_FEATURES), lambda i: (0, 0)),
        ],
        out_shape=[
            jax.ShapeDtypeStruct((BATCH, 1), jnp.int32),
            jax.ShapeDtypeStruct((1, N_CLUSTERS), jnp.float32),
            jax.ShapeDtypeStruct((N_CLUSTERS, N_FEATURES), jnp.float32),
        ],
        compiler_params=pltpu.CompilerParams(
            dimension_semantics=("arbitrary",),
        ),
    )(batch, centroids)
    return assign3.reshape(BATCH), counts2.reshape(N_CLUSTERS), sums


# speedup vs baseline: 1.0124x; 1.0124x over previous
import jax
import jax.numpy as jnp
from jax.experimental import pallas as pl
from jax.experimental.pallas import tpu as pltpu

N_CLUSTERS = 1024
N_FEATURES = 256
BATCH = 16384

TILE = 4096
NT = BATCH // TILE
W = 128
NCHUNK = N_CLUSTERS // W


def _fused_body(x_ref, c_ref, assign_ref, counts_ref, sums_ref):
    i = pl.program_id(0)
    x = x_ref[...]

    x2 = jnp.sum(x * x, axis=1, keepdims=True)

    best = None
    bcol = None
    for j in range(NCHUNK):
        cj = c_ref[pl.ds(j * W, W), :]
        c2j = jnp.sum(cj * cj, axis=1)[None, :]
        crossj = jax.lax.dot_general(
            x, cj, (((1,), (1,)), ((), ())),
            preferred_element_type=jnp.float32)
        dj = x2 + c2j - 2.0 * crossj
        if j == 0:
            best = dj
            bcol = jnp.zeros(dj.shape, jnp.float32)
        else:
            lt = dj < best
            best = jnp.where(lt, dj, best)
            bcol = jnp.where(lt, jnp.float32(j), bcol)

    dmin = jnp.min(best, axis=1, keepdims=True)
    lanef = jax.lax.broadcasted_iota(jnp.int32, best.shape, 1).astype(
        jnp.float32)
    big = jnp.float32(N_CLUSTERS)
    cand = jnp.where(best == dmin, bcol * jnp.float32(W) + lanef, big)
    assignf = jnp.min(cand, axis=1, keepdims=True)
    assign_ref[...] = assignf.astype(jnp.int32)

    oh_lane = cand == assignf

    @pl.when(i == 0)
    def _init():
        counts_ref[...] = jnp.zeros_like(counts_ref)
        sums_ref[...] = jnp.zeros_like(sums_ref)

    x_bf = x.astype(jnp.bfloat16)
    for j in range(NCHUNK):
        cond = oh_lane & (bcol == jnp.float32(j))
        ohj = jnp.where(cond, 1.0, 0.0)
        ohj_b = ohj.astype(jnp.bfloat16)
        counts_ref[:, pl.ds(j * W, W)] += jnp.sum(ohj, axis=0)[None, :]
        sums_ref[pl.ds(j * W, W), :] += jax.lax.dot_general(
            ohj_b, x_bf, (((0,), (0,)), ((), ())),
            preferred_element_type=jnp.float32)


@jax.jit
def kernel(batch, centroids):
    assign3, counts2, sums = pl.pallas_call(
        _fused_body,
        grid=(NT,),
        in_specs=[
            pl.BlockSpec((TILE, N_FEATURES), lambda i: (i, 0)),
            pl.BlockSpec((N_CLUSTERS, N_FEATURES), lambda i: (0, 0)),
        ],
        out_specs=[
            pl.BlockSpec((TILE, 1), lambda i: (i, 0)),
            pl.BlockSpec((1, N_CLUSTERS), lambda i: (0, 0)),
            pl.BlockSpec((N_CLUSTERS, N_FEATURES), lambda i: (0, 0)),
        ],
        out_shape=[
            jax.ShapeDtypeStruct((BATCH, 1), jnp.int32),
            jax.ShapeDtypeStruct((1, N_CLUSTERS), jnp.float32),
            jax.ShapeDtypeStruct((N_CLUSTERS, N_FEATURES), jnp.float32),
        ],
        compiler_params=pltpu.CompilerParams(
            dimension_semantics=("arbitrary",),
        ),
    )(batch, centroids)
    return assign3.reshape(BATCH), counts2.reshape(N_CLUSTERS), sums


# speedup vs baseline: 1.0798x; 1.0666x over previous
import jax
import jax.numpy as jnp
from jax.experimental import pallas as pl
from jax.experimental.pallas import tpu as pltpu

N_CLUSTERS = 1024
N_FEATURES = 256
BATCH = 16384

TILE = 4096
NT = BATCH // TILE
W = 128
NCHUNK = N_CLUSTERS // W


def _fused_body(x_ref, c_ref, assign_ref, counts_ref, sums_ref):
    i = pl.program_id(0)
    x = x_ref[...]

    x2 = jnp.sum(x * x, axis=1, keepdims=True)

    best = None
    bcol = None
    for j in range(NCHUNK):
        cj = c_ref[pl.ds(j * W, W), :]
        c2j = jnp.sum(cj * cj, axis=1)[None, :]
        crossj = jax.lax.dot_general(
            x, cj, (((1,), (1,)), ((), ())),
            preferred_element_type=jnp.float32)
        dj = x2 + c2j - 2.0 * crossj
        if j == 0:
            best = dj
            bcol = jnp.zeros(dj.shape, jnp.float32)
        else:
            lt = dj < best
            best = jnp.where(lt, dj, best)
            bcol = jnp.where(lt, jnp.float32(j), bcol)

    dmin = jnp.min(best, axis=1, keepdims=True)
    lanef = jax.lax.broadcasted_iota(jnp.int32, best.shape, 1).astype(
        jnp.float32)
    big = jnp.float32(N_CLUSTERS)
    cand = jnp.where(best == dmin, bcol * jnp.float32(W) + lanef, big)
    assignf = jnp.min(cand, axis=1, keepdims=True)
    assign_ref[...] = assignf.astype(jnp.int32)

    oh_lane = cand == assignf

    @pl.when(i == 0)
    def _init():
        counts_ref[...] = jnp.zeros_like(counts_ref)
        sums_ref[...] = jnp.zeros_like(sums_ref)

    x_bf = x.astype(jnp.bfloat16)
    merged = jnp.where(oh_lane, bcol, jnp.float32(-1))
    for j in range(NCHUNK):
        ohj = jnp.where(merged == jnp.float32(j), 1.0, 0.0)
        ohj_b = ohj.astype(jnp.bfloat16)
        counts_ref[:, pl.ds(j * W, W)] += jnp.sum(ohj, axis=0)[None, :]
        sums_ref[pl.ds(j * W, W), :] += jax.lax.dot_general(
            ohj_b, x_bf, (((0,), (0,)), ((), ())),
            preferred_element_type=jnp.float32)


@jax.jit
def kernel(batch, centroids):
    assign3, counts2, sums = pl.pallas_call(
        _fused_body,
        grid=(NT,),
        in_specs=[
            pl.BlockSpec((TILE, N_FEATURES), lambda i: (i, 0)),
            pl.BlockSpec((N_CLUSTERS, N_FEATURES), lambda i: (0, 0)),
        ],
        out_specs=[
            pl.BlockSpec((TILE, 1), lambda i: (i, 0)),
            pl.BlockSpec((1, N_CLUSTERS), lambda i: (0, 0)),
            pl.BlockSpec((N_CLUSTERS, N_FEATURES), lambda i: (0, 0)),
        ],
        out_shape=[
            jax.ShapeDtypeStruct((BATCH, 1), jnp.int32),
            jax.ShapeDtypeStruct((1, N_CLUSTERS), jnp.float32),
            jax.ShapeDtypeStruct((N_CLUSTERS, N_FEATURES), jnp.float32),
        ],
        compiler_params=pltpu.CompilerParams(
            dimension_semantics=("arbitrary",),
        ),
    )(batch, centroids)
    return assign3.reshape(BATCH), counts2.reshape(N_CLUSTERS), sums
